# baseline (speedup 1.0000x reference)
import functools

import jax
import jax.numpy as jnp
from jax import lax
from jax.experimental import pallas as pl
from jax.experimental.pallas import tpu as pltpu
from jax.experimental.pallas import tpu_sc as plsc

N_NODES_C = 100000
MAX_DEG_C = 64
NUM_ADJ_C = 2
BATCH_C = 16384
OUT_COLS = 32

_info = plsc.get_sparse_core_info()
_NC, _NS, _L = _info.num_cores, _info.num_subcores, _info.num_lanes
_NW = _NC * _NS
_B_PER_W = BATCH_C // _NW
_GATHER_CHUNK = 128
_N_CHUNKS = _B_PER_W // _GATHER_CHUNK


def _sc_gather(table, row_ids, cols):
    mesh = plsc.VectorSubcoreMesh(core_axis_name="c", subcore_axis_name="s")

    @functools.partial(
        pl.kernel,
        mesh=mesh,
        out_type=jax.ShapeDtypeStruct((BATCH_C, OUT_COLS), jnp.int32),
        scratch_types=[
            pltpu.VMEM((_B_PER_W,), jnp.int32),
            pltpu.VMEM((_B_PER_W, MAX_DEG_C), jnp.int32),
            pltpu.VMEM((OUT_COLS,), jnp.int32),
            pltpu.VMEM((_B_PER_W, OUT_COLS), jnp.int32),
            pltpu.SemaphoreType.DMA,
        ],
        compiler_params=pltpu.CompilerParams(
            needs_layout_passes=False, use_tc_tiling_on_sc=False
        ),
    )
    def k(table_hbm, ids_hbm, cols_hbm, out_hbm, idx_v, rows_v, cols_v, out_v, sem):
        wid = lax.axis_index("s") * _NC + lax.axis_index("c")
        base = wid * _B_PER_W
        pltpu.sync_copy(ids_hbm.at[pl.ds(base, _B_PER_W)], idx_v)
        pltpu.sync_copy(cols_hbm, cols_v)
        copies = []
        for ch in range(_N_CHUNKS):
            copies.append(
                pltpu.async_copy(
                    table_hbm.at[idx_v.at[pl.ds(ch * _GATHER_CHUNK, _GATHER_CHUNK)]],
                    rows_v.at[pl.ds(ch * _GATHER_CHUNK, _GATHER_CHUNK)],
                    sem,
                )
            )
        for c in copies:
            c.wait()
        cols_lo = cols_v[pl.ds(0, _L)]
        cols_hi = cols_v[pl.ds(_L, _L)]
        def per_row(b, carry):
            row = rows_v.at[b]
            lo = plsc.load_gather(row, [cols_lo])
            hi = plsc.load_gather(row, [cols_hi])
            out_v[b, pl.ds(0, _L)] = lo
            out_v[b, pl.ds(_L, _L)] = hi
            return carry

        lax.fori_loop(0, _B_PER_W, per_row, 0, unroll=4)
        pltpu.sync_copy(out_v, out_hbm.at[pl.ds(base, _B_PER_W)])

    return k(table, row_ids, cols)


def kernel(adj_info, ids, num_samples, aid):
    table = adj_info.reshape(NUM_ADJ_C * N_NODES_C, MAX_DEG_C)
    row_ids = (ids + aid * N_NODES_C).astype(jnp.int32)
    perm = jax.random.permutation(jax.random.key(42), MAX_DEG_C)
    start = (num_samples - OUT_COLS).astype(jnp.int32) if hasattr(
        num_samples, "astype") else jnp.int32(num_samples - OUT_COLS)
    cols = lax.dynamic_slice(perm.astype(jnp.int32), (start,), (OUT_COLS,))
    return _sc_gather(table, row_ids, cols)


# speedup vs baseline: 1.4793x; 1.4793x over previous
import functools

import jax
import jax.numpy as jnp
from jax import lax
from jax.experimental import pallas as pl
from jax.experimental.pallas import tpu as pltpu
from jax.experimental.pallas import tpu_sc as plsc

N_NODES_C = 100000
MAX_DEG_C = 64
NUM_ADJ_C = 2
BATCH_C = 16384
OUT_COLS = 32

_info = plsc.get_sparse_core_info()
_NC, _NS, _L = _info.num_cores, _info.num_subcores, _info.num_lanes
_NW = _NC * _NS
_B_PER_W = BATCH_C // _NW
_RING = 16


def _sc_gather(table, row_ids, cols):
    mesh = plsc.VectorSubcoreMesh(core_axis_name="c", subcore_axis_name="s")

    @functools.partial(
        pl.kernel,
        mesh=mesh,
        out_type=jax.ShapeDtypeStruct((BATCH_C, OUT_COLS), jnp.int32),
        scratch_types=[
            pltpu.VMEM((_B_PER_W,), jnp.int32),
            pltpu.VMEM((_RING, MAX_DEG_C), jnp.int32),
            pltpu.VMEM((OUT_COLS,), jnp.int32),
            pltpu.VMEM((_B_PER_W, OUT_COLS), jnp.int32),
            [pltpu.SemaphoreType.DMA] * _RING,
        ],
        compiler_params=pltpu.CompilerParams(needs_layout_passes=False),
    )
    def k(table_hbm, ids_hbm, cols_hbm, out_hbm, idx_v, ring_v, cols_v, out_v,
          sems):
        wid = lax.axis_index("s") * _NC + lax.axis_index("c")
        base = wid * _B_PER_W
        pltpu.sync_copy(ids_hbm.at[pl.ds(base, _B_PER_W)], idx_v)
        pltpu.sync_copy(cols_hbm, cols_v)
        cols_lo = cols_v[pl.ds(0, _L)]
        cols_hi = cols_v[pl.ds(_L, _L)]

        def fetch(rid, slot):
            pltpu.async_copy(
                table_hbm.at[pl.ds(rid, 1)],
                ring_v.at[pl.ds(slot, 1)],
                sems[slot],
            )

        def drain(slot):
            pltpu.make_async_copy(
                table_hbm.at[pl.ds(0, 1)],
                ring_v.at[pl.ds(slot, 1)],
                sems[slot],
            ).wait()

        idvec0 = idx_v[pl.ds(0, _L)]
        for p in range(_RING):
            fetch(idvec0[p], p)

        def per_group(g, carry):
            nxt = idx_v[pl.ds(lax.min((g + 1) * _L, _B_PER_W - _L), _L)]
            for p in range(_RING):
                b = g * _RING + p
                drain(p)
                svec = jnp.full((_L,), p, dtype=jnp.int32)
                lo = plsc.load_gather(ring_v, [svec, cols_lo])
                hi = plsc.load_gather(ring_v, [svec, cols_hi])
                out_v[b, pl.ds(0, _L)] = lo
                out_v[b, pl.ds(_L, _L)] = hi

                @pl.when(g + 1 < _B_PER_W // _RING)
                def _():
                    fetch(nxt[p], p)

            return carry

        lax.fori_loop(0, _B_PER_W // _RING, per_group, 0)
        pltpu.sync_copy(out_v, out_hbm.at[pl.ds(base, _B_PER_W)])

    return k(table, row_ids, cols)


def kernel(adj_info, ids, num_samples, aid):
    table = adj_info.reshape(NUM_ADJ_C * N_NODES_C, MAX_DEG_C)
    row_ids = (ids + aid * N_NODES_C).astype(jnp.int32)
    perm = jax.random.permutation(jax.random.key(42), MAX_DEG_C)
    start = (num_samples - OUT_COLS).astype(jnp.int32) if hasattr(
        num_samples, "astype") else jnp.int32(num_samples - OUT_COLS)
    cols = lax.dynamic_slice(perm.astype(jnp.int32), (start,), (OUT_COLS,))
    return _sc_gather(table, row_ids, cols)


# speedup vs baseline: 1.4832x; 1.0026x over previous
import functools

import jax
import jax.numpy as jnp
from jax import lax
from jax.experimental import pallas as pl
from jax.experimental.pallas import tpu as pltpu
from jax.experimental.pallas import tpu_sc as plsc

N_NODES_C = 100000
MAX_DEG_C = 64
NUM_ADJ_C = 2
BATCH_C = 16384
OUT_COLS = 32

_info = plsc.get_sparse_core_info()
_NC, _NS, _L = _info.num_cores, _info.num_subcores, _info.num_lanes
_NW = _NC * _NS
_B_PER_W = BATCH_C // _NW
_RING = 16


def _sc_gather(table, row_ids, cols):
    mesh = plsc.VectorSubcoreMesh(core_axis_name="c", subcore_axis_name="s")

    @functools.partial(
        pl.kernel,
        mesh=mesh,
        out_type=jax.ShapeDtypeStruct((BATCH_C, OUT_COLS), jnp.int32),
        scratch_types=[
            pltpu.VMEM((_B_PER_W,), jnp.int32),
            pltpu.VMEM((_RING, MAX_DEG_C), jnp.int32),
            pltpu.VMEM((OUT_COLS,), jnp.int32),
            pltpu.VMEM((_B_PER_W, OUT_COLS), jnp.int32),
            [pltpu.SemaphoreType.DMA] * _RING,
        ],
        compiler_params=pltpu.CompilerParams(
            needs_layout_passes=False, use_tc_tiling_on_sc=True
        ),
    )
    def k(table_hbm, ids_hbm, cols_hbm, out_hbm, idx_v, ring_v, cols_v, out_v,
          sems):
        wid = lax.axis_index("s") * _NC + lax.axis_index("c")
        base = wid * _B_PER_W
        pltpu.sync_copy(ids_hbm.at[pl.ds(base, _B_PER_W)], idx_v)
        pltpu.sync_copy(cols_hbm, cols_v)
        cols_lo = cols_v[pl.ds(0, _L)]
        cols_hi = cols_v[pl.ds(_L, _L)]

        def fetch(rid, slot):
            pltpu.async_copy(
                table_hbm.at[pl.ds(rid, 1)],
                ring_v.at[pl.ds(slot, 1)],
                sems[slot],
            )

        def drain(slot):
            pltpu.make_async_copy(
                table_hbm.at[pl.ds(0, 1)],
                ring_v.at[pl.ds(slot, 1)],
                sems[slot],
            ).wait()

        idvec0 = idx_v[pl.ds(0, _L)]
        for p in range(_RING):
            fetch(idvec0[p], p)

        def per_group(g, carry):
            nxt = idx_v[pl.ds(lax.min((g + 1) * _L, _B_PER_W - _L), _L)]
            for p in range(_RING):
                b = g * _RING + p
                drain(p)
                svec = jnp.full((_L,), p, dtype=jnp.int32)
                lo = plsc.load_gather(ring_v, [svec, cols_lo])
                hi = plsc.load_gather(ring_v, [svec, cols_hi])
                out_v[b, pl.ds(0, _L)] = lo
                out_v[b, pl.ds(_L, _L)] = hi

                @pl.when(g + 1 < _B_PER_W // _RING)
                def _():
                    fetch(nxt[p], p)

            return carry

        lax.fori_loop(0, _B_PER_W // _RING, per_group, 0)
        pltpu.sync_copy(out_v, out_hbm.at[pl.ds(base, _B_PER_W)])

    return k(table, row_ids, cols)


def kernel(adj_info, ids, num_samples, aid):
    table = adj_info.reshape(NUM_ADJ_C * N_NODES_C, MAX_DEG_C)
    row_ids = (ids + aid * N_NODES_C).astype(jnp.int32)
    perm = jax.random.permutation(jax.random.key(42), MAX_DEG_C)
    start = (num_samples - OUT_COLS).astype(jnp.int32) if hasattr(
        num_samples, "astype") else jnp.int32(num_samples - OUT_COLS)
    cols = lax.dynamic_slice(perm.astype(jnp.int32), (start,), (OUT_COLS,))
    return _sc_gather(table, row_ids, cols)


# speedup vs baseline: 3.4989x; 2.3591x over previous
import functools

import jax
import jax.numpy as jnp
from jax import lax
from jax.experimental import pallas as pl
from jax.experimental.pallas import tpu as pltpu
from jax.experimental.pallas import tpu_sc as plsc

N_NODES_C = 100000
MAX_DEG_C = 64
NUM_ADJ_C = 2
BATCH_C = 16384
OUT_COLS = 32

_info = plsc.get_sparse_core_info()
_NC, _NS, _L = _info.num_cores, _info.num_subcores, _info.num_lanes
_NW = _NC * _NS
_SEG = BATCH_C // 2


def _sc_gather(table_t, ids, row_list):
    mesh = plsc.VectorSubcoreMesh(core_axis_name="c", subcore_axis_name="s")

    @functools.partial(
        pl.kernel,
        mesh=mesh,
        out_type=jax.ShapeDtypeStruct((OUT_COLS, BATCH_C), jnp.int32),
        scratch_types=[
            pltpu.VMEM((1, N_NODES_C), jnp.int32),
            pltpu.VMEM((OUT_COLS,), jnp.int32),
            pltpu.VMEM((_SEG,), jnp.int32),
            pltpu.VMEM((_SEG,), jnp.int32),
            pltpu.SemaphoreType.DMA,
        ],
        compiler_params=pltpu.CompilerParams(
            needs_layout_passes=False, use_tc_tiling_on_sc=True
        ),
    )
    def k(table_hbm, ids_hbm, rows_hbm, out_hbm, row_v, rl_v, ids_v, col_v, sem):
        w = lax.axis_index("s") * _NC + lax.axis_index("c")
        pltpu.sync_copy(rows_hbm, rl_v)
        lane = lax.rem(w, _L)
        half = lax.div(w, _L)
        vec = jnp.where(
            jnp.full((_L,), half, dtype=jnp.int32) == 0,
            rl_v[pl.ds(0, _L)],
            rl_v[pl.ds(_L, _L)],
        )
        lanes = lax.iota(jnp.int32, _L)
        masked = jnp.where(lanes == jnp.full((_L,), lane, dtype=jnp.int32),
                           vec, jnp.zeros((_L,), jnp.int32))
        r = jnp.max(masked)
        pltpu.async_copy(table_hbm.at[pl.ds(r, 1)], row_v, sem).wait()
        row_flat = row_v.at[0]

        for seg in range(BATCH_C // _SEG):
            pltpu.sync_copy(ids_hbm.at[pl.ds(seg * _SEG, _SEG)], ids_v)

            def per_vec(i, carry):
                iv = ids_v[pl.ds(i * _L, _L)]
                col_v[pl.ds(i * _L, _L)] = plsc.load_gather(row_flat, [iv])
                return carry

            lax.fori_loop(0, _SEG // _L, per_vec, 0, unroll=8)
            pltpu.sync_copy(
                col_v, out_hbm.at[w, pl.ds(seg * _SEG, _SEG)]
            )

    return k(table_t, ids, row_list)


def kernel(adj_info, ids, num_samples, aid):
    table_t = adj_info.transpose(0, 2, 1).reshape(
        NUM_ADJ_C * MAX_DEG_C, N_NODES_C)
    perm = jax.random.permutation(jax.random.key(42), MAX_DEG_C)
    start = (num_samples - OUT_COLS).astype(jnp.int32) if hasattr(
        num_samples, "astype") else jnp.int32(num_samples - OUT_COLS)
    cols = lax.dynamic_slice(perm.astype(jnp.int32), (start,), (OUT_COLS,))
    row_list = (cols + aid * MAX_DEG_C).astype(jnp.int32)
    out_t = _sc_gather(table_t, ids.astype(jnp.int32), row_list)
    return out_t.T


# speedup vs baseline: 3.9901x; 1.1404x over previous
import functools

import jax
import jax.numpy as jnp
import numpy as np
from jax import lax
from jax.experimental import pallas as pl
from jax.experimental.pallas import tpu as pltpu
from jax.experimental.pallas import tpu_sc as plsc

N_NODES_C = 100000
MAX_DEG_C = 64
NUM_ADJ_C = 2
BATCH_C = 16384
OUT_COLS = 32

_info = plsc.get_sparse_core_info()
_NC, _NS, _L = _info.num_cores, _info.num_subcores, _info.num_lanes
_NW = _NC * _NS
_SEG = BATCH_C // 2


_PERM = np.asarray(jax.random.permutation(jax.random.key(42), MAX_DEG_C),
                   dtype=np.int32)


def _sc_gather(table_t, ids, row_list):
    mesh = plsc.VectorSubcoreMesh(core_axis_name="c", subcore_axis_name="s")

    @functools.partial(
        pl.kernel,
        mesh=mesh,
        out_type=jax.ShapeDtypeStruct((OUT_COLS, BATCH_C), jnp.int32),
        scratch_types=[
            pltpu.VMEM((1, N_NODES_C), jnp.int32),
            pltpu.VMEM((OUT_COLS,), jnp.int32),
            pltpu.VMEM((_SEG,), jnp.int32),
            pltpu.VMEM((_SEG,), jnp.int32),
            pltpu.VMEM((_SEG,), jnp.int32),
            pltpu.SemaphoreType.DMA,
            pltpu.SemaphoreType.DMA,
            pltpu.SemaphoreType.DMA,
        ],
        compiler_params=pltpu.CompilerParams(
            needs_layout_passes=False, use_tc_tiling_on_sc=True
        ),
    )
    def k(table_hbm, ids_hbm, rows_hbm, out_hbm, row_v, rl_v, ids_v, col_a,
          col_b, sem_r, sem_i, sem_o):
        w = lax.axis_index("s") * _NC + lax.axis_index("c")
        pltpu.sync_copy(rows_hbm, rl_v)
        lane = lax.rem(w, _L)
        half = lax.div(w, _L)
        vec = jnp.where(
            jnp.full((_L,), half, dtype=jnp.int32) == 0,
            rl_v[pl.ds(0, _L)],
            rl_v[pl.ds(_L, _L)],
        )
        lanes = lax.iota(jnp.int32, _L)
        masked = jnp.where(lanes == jnp.full((_L,), lane, dtype=jnp.int32),
                           vec, jnp.zeros((_L,), jnp.int32))
        r = jnp.max(masked)
        row_dma = pltpu.async_copy(table_hbm.at[pl.ds(r, 1)], row_v, sem_r)
        ids0_dma = pltpu.async_copy(ids_hbm.at[pl.ds(0, _SEG)], ids_v, sem_i)
        row_flat = row_v.at[0]
        row_dma.wait()
        ids0_dma.wait()

        def per_vec(i, carry):
            iv = ids_v[pl.ds(i * _L, _L)]
            col_a[pl.ds(i * _L, _L)] = plsc.load_gather(row_flat, [iv])
            return carry

        lax.fori_loop(0, _SEG // _L, per_vec, 0, unroll=8)
        out0_dma = pltpu.async_copy(col_a, out_hbm.at[w, pl.ds(0, _SEG)],
                                    sem_o)
        pltpu.sync_copy(ids_hbm.at[pl.ds(_SEG, _SEG)], ids_v)

        def per_vec2(i, carry):
            iv = ids_v[pl.ds(i * _L, _L)]
            col_b[pl.ds(i * _L, _L)] = plsc.load_gather(row_flat, [iv])
            return carry

        lax.fori_loop(0, _SEG // _L, per_vec2, 0, unroll=8)
        out0_dma.wait()
        pltpu.sync_copy(col_b, out_hbm.at[w, pl.ds(_SEG, _SEG)])

    return k(table_t, ids, row_list)


def kernel(adj_info, ids, num_samples, aid):
    table_t = adj_info.transpose(0, 2, 1).reshape(
        NUM_ADJ_C * MAX_DEG_C, N_NODES_C)
    perm = jnp.asarray(_PERM)
    start = (num_samples - OUT_COLS).astype(jnp.int32) if hasattr(
        num_samples, "astype") else jnp.int32(num_samples - OUT_COLS)
    cols = lax.dynamic_slice(perm, (start,), (OUT_COLS,))
    row_list = (cols + aid * MAX_DEG_C).astype(jnp.int32)
    out_t = _sc_gather(table_t, ids.astype(jnp.int32), row_list)
    return out_t.T


# speedup vs baseline: 4.9126x; 1.2312x over previous
import functools

import jax
import jax.numpy as jnp
import numpy as np
from jax import lax
from jax.experimental import pallas as pl
from jax.experimental.pallas import tpu as pltpu
from jax.experimental.pallas import tpu_sc as plsc

N_NODES_C = 100000
MAX_DEG_C = 64
NUM_ADJ_C = 2
BATCH_C = 16384
OUT_COLS = 32

_info = plsc.get_sparse_core_info()
_NC, _NS, _L = _info.num_cores, _info.num_subcores, _info.num_lanes
_NW = _NC * _NS
_SEG = BATCH_C // 2
_CHUNK = 128
_RING_G = 8


_PERM = np.array(
    [35, 45, 31, 63, 7, 4, 29, 44, 16, 58, 37, 19, 61, 2, 34, 5,
     30, 42, 3, 39, 56, 22, 6, 54, 18, 10, 11, 53, 32, 15, 49, 50,
     20, 43, 8, 24, 9, 40, 59, 25, 13, 52, 62, 60, 47, 33, 14, 17,
     38, 23, 0, 41, 21, 26, 57, 1, 28, 48, 36, 55, 51, 27, 12, 46],
    dtype=np.int32)


def _sc_gather(table_t, ids, row_list):
    mesh = plsc.VectorSubcoreMesh(core_axis_name="c", subcore_axis_name="s")

    @functools.partial(
        pl.kernel,
        mesh=mesh,
        out_type=jax.ShapeDtypeStruct((OUT_COLS, BATCH_C), jnp.int32),
        scratch_types=[
            pltpu.VMEM((1, N_NODES_C), jnp.int32),
            pltpu.VMEM((OUT_COLS,), jnp.int32),
            pltpu.VMEM((_SEG,), jnp.int32),
            pltpu.VMEM((_SEG,), jnp.int32),
            pltpu.VMEM((_SEG,), jnp.int32),
            pltpu.SemaphoreType.DMA,
            pltpu.SemaphoreType.DMA,
            pltpu.SemaphoreType.DMA,
            pltpu.SemaphoreType.DMA,
        ],
        compiler_params=pltpu.CompilerParams(
            needs_layout_passes=False, use_tc_tiling_on_sc=True
        ),
    )
    def k(table_hbm, ids_hbm, rows_hbm, out_hbm, row_v, rl_v, ids_v, col_a,
          col_b, sem_r, sem_i, sem_o, sem_g):
        w = lax.axis_index("s") * _NC + lax.axis_index("c")
        pltpu.sync_copy(rows_hbm, rl_v)
        lane = lax.rem(w, _L)
        half = lax.div(w, _L)
        vec = jnp.where(
            jnp.full((_L,), half, dtype=jnp.int32) == 0,
            rl_v[pl.ds(0, _L)],
            rl_v[pl.ds(_L, _L)],
        )
        lanes = lax.iota(jnp.int32, _L)
        masked = jnp.where(lanes == jnp.full((_L,), lane, dtype=jnp.int32),
                           vec, jnp.zeros((_L,), jnp.int32))
        r = jnp.max(masked)
        row_dma = pltpu.async_copy(table_hbm.at[pl.ds(r, 1)], row_v, sem_r)
        ids0_dma = pltpu.async_copy(ids_hbm.at[pl.ds(0, _SEG)], ids_v, sem_i)
        row_flat = row_v.at[0]
        row_dma.wait()
        ids0_dma.wait()

        def gather_seg(col_v):
            @plsc.parallel_loop(0, _SEG // _L, 1, unroll=8)
            def _(i):
                iv = ids_v[pl.ds(i * _L, _L)]
                col_v[pl.ds(i * _L, _L)] = plsc.load_gather(row_flat, [iv])

        gather_seg(col_a)
        out0_dma = pltpu.async_copy(col_a, out_hbm.at[w, pl.ds(0, _SEG)],
                                    sem_o)
        pltpu.sync_copy(ids_hbm.at[pl.ds(_SEG, _SEG)], ids_v)
        gather_seg(col_b)
        out0_dma.wait()
        pltpu.sync_copy(col_b, out_hbm.at[w, pl.ds(_SEG, _SEG)])

    return k(table_t, ids, row_list)


def kernel(adj_info, ids, num_samples, aid):
    table_t = adj_info.transpose(0, 2, 1).reshape(
        NUM_ADJ_C * MAX_DEG_C, N_NODES_C)
    perm = jnp.asarray(_PERM)
    start = (num_samples - OUT_COLS).astype(jnp.int32) if hasattr(
        num_samples, "astype") else jnp.int32(num_samples - OUT_COLS)
    cols = lax.dynamic_slice(perm, (start,), (OUT_COLS,))
    row_list = (cols + aid * MAX_DEG_C).astype(jnp.int32)
    out_t = _sc_gather(table_t, ids.astype(jnp.int32), row_list)
    return out_t.T
